# Initial kernel scaffold; baseline (speedup 1.0000x reference)
import jax
import jax.numpy as jnp
from jax.experimental import pallas as pl


def kernel(x, weight):
    raise NotImplementedError("write your pallas kernel here")


# speedup vs baseline: 1.8766x; 1.8766x over previous
import functools

import jax
import jax.numpy as jnp
from jax import lax
from jax.experimental import pallas as pl
from jax.experimental.pallas import tpu as pltpu
from jax.experimental.pallas import tpu_sc as plsc

NUM_CORES = 2
NUM_SUBCORES = 16
NW = NUM_CORES * NUM_SUBCORES

CHUNK = 128
NBUF = 8


@functools.lru_cache(maxsize=None)
def _build_gather(B: int, V: int, D: int):
    assert B % (NW * CHUNK) == 0
    bpw = B // NW
    nchunk = bpw // CHUNK
    assert nchunk > NBUF and nchunk % NBUF == 0

    mesh = plsc.VectorSubcoreMesh(core_axis_name="c", subcore_axis_name="s")

    @functools.partial(
        pl.kernel,
        out_type=jax.ShapeDtypeStruct((B, D), jnp.float32),
        mesh=mesh,
        scratch_types=[
            pltpu.VMEM((bpw,), jnp.int32),
            pltpu.VMEM((NBUF, CHUNK, D), jnp.float32),
        ] + [pltpu.SemaphoreType.DMA] * NBUF,
        compiler_params=pltpu.CompilerParams(use_tc_tiling_on_sc=False),
    )
    def gather_kernel(idx_hbm, table_hbm, out_hbm, idx_v, rows_v, *sems):
        wid = lax.axis_index("s") * NUM_CORES + lax.axis_index("c")
        base = wid * bpw
        pltpu.sync_copy(idx_hbm.at[pl.ds(base, bpw)], idx_v)

        def issue(chunk, b):
            pltpu.async_copy(
                table_hbm.at[idx_v.at[pl.ds(chunk * CHUNK, CHUNK)]],
                rows_v.at[b],
                sems[b],
            )

        def drain(b):
            pltpu.make_async_copy(
                table_hbm.at[pl.ds(0, CHUNK)], rows_v.at[b], sems[b]
            ).wait()

        def store(chunk, b):
            pltpu.sync_copy(
                rows_v.at[b], out_hbm.at[pl.ds(base + chunk * CHUNK, CHUNK)]
            )

        for b in range(NBUF):
            issue(b, b)

        @pl.loop(0, (nchunk - NBUF) // NBUF)
        def _(q):
            g0 = q * NBUF
            for b in range(NBUF):
                drain(b)
                store(g0 + b, b)
                issue(g0 + b + NBUF, b)

        for b in range(NBUF):
            drain(b)
            store(nchunk - NBUF + b, b)

    return gather_kernel


def kernel(x, weight):
    V, D = weight.shape
    idx = x.reshape(-1)
    out = _build_gather(idx.shape[0], V, D)(idx, weight)
    return out.reshape(x.shape + (D,))
